# baseline (speedup 1.0000x reference)
import functools

import jax
import jax.numpy as jnp
from jax import lax
from jax.experimental import pallas as pl
from jax.experimental.pallas import tpu as pltpu
from jax.experimental.pallas import tpu_sc as plsc

_NC = 2
_NS = 16
_NW = _NC * _NS
_IPD = 128
_DPC = 2


def _sc_gather_grp(table_grp, gidx3, sidx3, n_rows):
    rpw = n_rows // _NW
    n_dma = rpw // _IPD
    chunk_rows = _IPD * _DPC
    n_chunks = n_dma // _DPC
    mesh = plsc.VectorSubcoreMesh(core_axis_name="c", subcore_axis_name="s")

    @functools.partial(
        pl.kernel,
        out_type=jax.ShapeDtypeStruct((n_rows // 8, 128), jnp.float32),
        mesh=mesh,
        compiler_params=pltpu.CompilerParams(
            use_tc_tiling_on_sc=True, needs_layout_passes=False),
        scratch_types=[
            pltpu.VMEM((n_dma, _IPD), jnp.int32),
            pltpu.VMEM((n_dma, _IPD), jnp.int32),
            pltpu.VMEM((chunk_rows, 128), jnp.float32),
            pltpu.VMEM((chunk_rows // 8, 128), jnp.float32),
            pltpu.SemaphoreType.DMA,
        ],
    )
    def gather_k(table_hbm, gidx_hbm, sidx_hbm, out_hbm, gidx_v, sidx_v,
                 grp_v, out_v, gsem):
        wid = lax.axis_index("s") * _NC + lax.axis_index("c")
        row0 = wid * rpw
        pltpu.sync_copy(gidx_hbm.at[wid], gidx_v)
        pltpu.sync_copy(sidx_hbm.at[wid], sidx_v)
        lanes = lax.iota(jnp.int32, 16)

        def chunk_body(c, carry):
            copies = []
            for m in range(_DPC):
                cp = pltpu.async_copy(
                    table_hbm.at[gidx_v.at[c * _DPC + m]],
                    grp_v.at[pl.ds(m * _IPD, _IPD)],
                    gsem,
                )
                copies.append(cp)
            for cp in copies:
                cp.wait()

            def tile_body(m, carry2):
                for k in range(8):
                    s = sidx_v[c * _DPC + m, pl.ds(k * 16, 16)] & 7
                    rows = m * 128 + k * 16 + lanes
                    base = s * 16
                    for d in range(16):
                        vals = plsc.load_gather(grp_v, [rows, base + d])
                        fj = rows * 16 + d
                        plsc.store_scatter(out_v, [fj >> 7, fj & 127], vals)
                return carry2

            lax.fori_loop(0, _DPC, tile_body, 0)
            off = pl.multiple_of((row0 + c * chunk_rows) // 8, chunk_rows // 8)
            pltpu.sync_copy(out_v, out_hbm.at[pl.ds(off, chunk_rows // 8)])
            return carry

        lax.fori_loop(0, n_chunks, chunk_body, 0)

    return gather_k(table_grp, gidx3, sidx3)


def _tc_mlp(x, w1, b1, wp, bp, bt):
    b, d_in = x.shape
    h = w1.shape[1]

    def mlp_k(x_ref, w1_ref, b1_ref, wp_ref, bp_ref, o_ref):
        acc = jnp.dot(x_ref[...], w1_ref[...], preferred_element_type=jnp.float32)
        acc = jnp.maximum(acc + b1_ref[...], 0.0)
        out = jnp.dot(acc, wp_ref[...], preferred_element_type=jnp.float32)
        o_ref[...] = jax.nn.sigmoid(out + bp_ref[...])

    return pl.pallas_call(
        mlp_k,
        grid=(b // bt,),
        in_specs=[
            pl.BlockSpec((bt, d_in), lambda i: (i, 0)),
            pl.BlockSpec((d_in, h), lambda i: (0, 0)),
            pl.BlockSpec((1, h), lambda i: (0, 0)),
            pl.BlockSpec((h, 1), lambda i: (0, 0)),
            pl.BlockSpec((1, 1), lambda i: (0, 0)),
        ],
        out_specs=pl.BlockSpec((bt, 1), lambda i: (i, 0)),
        out_shape=jax.ShapeDtypeStruct((b, 1), jnp.float32),
    )(x, w1, b1, wp, bp)


def kernel(x_categorical, tables, W1, b1, Wp, bp):
    f, v, d = tables.shape
    b = x_categorical.shape[0]
    h = W1.shape[1]
    n_rows = b * f
    flat_idx = x_categorical + (jnp.arange(f, dtype=jnp.int32) * v)[None, :]
    gidx3 = (flat_idx >> 3).reshape(_NW, (n_rows // _NW) // _IPD, _IPD)
    sidx3 = flat_idx.reshape(_NW, (n_rows // _NW) // _IPD, _IPD)
    table_grp = tables.reshape(f * v // 8, 8 * d)
    emb8 = _sc_gather_grp(table_grp, gidx3, sidx3, n_rows)
    x = emb8.reshape(b, f * d)
    return _tc_mlp(x, W1, b1.reshape(1, h), Wp, bp.reshape(1, 1), 2048)
